# Initial kernel scaffold; baseline (speedup 1.0000x reference)
import jax
import jax.numpy as jnp
from jax.experimental import pallas as pl


def kernel(edge_index, edge_type, emb, basis1, comp1, root1, bias1, ln1_g, ln1_b, basis2, comp2, root2, bias2, ln2_g, ln2_b):
    raise NotImplementedError("write your pallas kernel here")


# speedup vs baseline: 8.1704x; 8.1704x over previous
import dataclasses
import functools

import jax
import jax.numpy as jnp
from jax import lax
from jax.experimental import pallas as pl
from jax.experimental.pallas import tpu as pltpu
from jax.experimental.pallas import tpu_sc as plsc

N = 10000
E = 320000
D = 128
R = 8
B = 4

NC = 2
NS = 16
NW = NC * NS
L = 16

K = 80
EPT = E // NW
WINS = EPT // K
CW = 16
SEG = R * N

_mesh = functools.partial(
    plsc.VectorSubcoreMesh, core_axis_name="c", subcore_axis_name="s"
)

_SC_PARAMS = pltpu.CompilerParams()
if "needs_layout_passes" in pltpu.CompilerParams.__dataclass_fields__:
    _SC_PARAMS = dataclasses.replace(_SC_PARAMS, needs_layout_passes=False)


def _sc_counts(dst, et):
    NP = 10240
    zrows = 128

    @functools.partial(
        pl.kernel,
        mesh=_mesh(),
        out_type=jax.ShapeDtypeStruct((NC, NP, D), jnp.float32),
        compiler_params=_SC_PARAMS,
        scratch_types=[
            pltpu.VMEM((K,), jnp.int32),
            pltpu.VMEM((K,), jnp.int32),
            pltpu.VMEM((K, D), jnp.float32),
            pltpu.VMEM((zrows, D), jnp.float32),
            pltpu.VMEM_SHARED((NP, D), jnp.float32),
        ],
    )
    def k(dst_hbm, et_hbm, out_hbm, dst_v, et_v, rows_v, z_v, acc_sh):
        c = lax.axis_index("c")
        s = lax.axis_index("s")

        @pl.loop(0, K)
        def _(i):
            for j in range(D // L):
                rows_v[i, pl.ds(j * L, L)] = jnp.zeros((L,), jnp.float32)

        @pl.loop(0, zrows)
        def _(i):
            for j in range(D // L):
                z_v[i, pl.ds(j * L, L)] = jnp.zeros((L,), jnp.float32)

        rpt = NP // NS

        @pl.loop(0, rpt // zrows)
        def _(i):
            pltpu.sync_copy(z_v, acc_sh.at[pl.ds(s * rpt + i * zrows, zrows)])

        plsc.subcore_barrier()
        base = (c * NS + s) * EPT
        ones16 = jnp.ones((L,), jnp.float32)
        zeros16 = jnp.zeros((L,), jnp.float32)
        iota16 = lax.broadcasted_iota(jnp.int32, (L,), 0)

        @pl.loop(0, WINS)
        def _(w):
            o = base + w * K
            pltpu.sync_copy(dst_hbm.at[pl.ds(o, K)], dst_v)
            pltpu.sync_copy(et_hbm.at[pl.ds(o, K)], et_v)
            for j in range(K // L):
                rid = iota16 + (j * L)
                col = et_v[pl.ds(j * L, L)] * L
                plsc.store_scatter(rows_v, [rid, col], ones16)
            pltpu.sync_copy(rows_v, acc_sh.at[dst_v], add=True)
            for j in range(K // L):
                rid = iota16 + (j * L)
                col = et_v[pl.ds(j * L, L)] * L
                plsc.store_scatter(rows_v, [rid, col], zeros16)

        plsc.subcore_barrier()

        @pl.loop(0, rpt // zrows)
        def _(i):
            pltpu.sync_copy(
                acc_sh.at[pl.ds(s * rpt + i * zrows, zrows)],
                out_hbm.at[c].at[pl.ds(s * rpt + i * zrows, zrows)],
            )

    return k(dst, et)


def _tc_inv(cgrid):
    def body(c_ref, o_ref):
        cnt = c_ref[0] + c_ref[1]
        o_ref[...] = 1.0 / jnp.maximum(cnt, 1.0)

    return pl.pallas_call(
        body,
        out_shape=jax.ShapeDtypeStruct((N, R), jnp.float32),
    )(cgrid)


def _sc_scale(dst, et, inv):
    @functools.partial(
        pl.kernel,
        mesh=_mesh(),
        out_type=jax.ShapeDtypeStruct((E,), jnp.float32),
        compiler_params=_SC_PARAMS,
        scratch_types=[
            pltpu.VMEM((SEG,), jnp.float32),
            pltpu.VMEM((K,), jnp.int32),
            pltpu.VMEM((K,), jnp.int32),
            pltpu.VMEM((K,), jnp.float32),
        ],
    )
    def k(dst_hbm, et_hbm, inv_hbm, out_hbm, inv_v, dst_v, et_v, scl_v):
        c = lax.axis_index("c")
        s = lax.axis_index("s")
        pltpu.sync_copy(inv_hbm, inv_v)
        base = (c * NS + s) * EPT

        @pl.loop(0, WINS)
        def _(w):
            o = base + w * K
            pltpu.sync_copy(dst_hbm.at[pl.ds(o, K)], dst_v)
            pltpu.sync_copy(et_hbm.at[pl.ds(o, K)], et_v)
            for j in range(K // L):
                d16 = dst_v[pl.ds(j * L, L)]
                t16 = et_v[pl.ds(j * L, L)]
                scl_v[pl.ds(j * L, L)] = plsc.load_gather(inv_v, [d16 * R + t16])
            pltpu.sync_copy(scl_v, out_hbm.at[pl.ds(o, K)])

    return k(dst, et, inv)


def _sc_edge(src, et, dst, scl, hr):
    zrows = 128
    npad = 10240

    @functools.partial(
        pl.kernel,
        mesh=_mesh(),
        out_type=jax.ShapeDtypeStruct((NC, npad, D), jnp.float32),
        compiler_params=_SC_PARAMS,
        scratch_types=[
            pltpu.VMEM((K,), jnp.int32),
            pltpu.VMEM((K,), jnp.int32),
            pltpu.VMEM((K,), jnp.int32),
            pltpu.VMEM((K,), jnp.int32),
            pltpu.VMEM((K,), jnp.float32),
            pltpu.VMEM((K, D), jnp.float32),
            pltpu.VMEM((zrows, D), jnp.float32),
            pltpu.VMEM_SHARED((npad, D), jnp.float32),
        ],
    )
    def k(src_hbm, et_hbm, dst_hbm, scl_hbm, hr_hbm, out_hbm,
          src_v, et_v, dst_v, gidx_v, scl_v, rows_v, z_v, acc_sh):
        c = lax.axis_index("c")
        s = lax.axis_index("s")

        @pl.loop(0, zrows)
        def _(i):
            for j in range(D // L):
                z_v[i, pl.ds(j * L, L)] = jnp.zeros((L,), jnp.float32)

        rpt = npad // NS

        @pl.loop(0, rpt // zrows)
        def _(i):
            pltpu.sync_copy(z_v, acc_sh.at[pl.ds(s * rpt + i * zrows, zrows)])

        plsc.subcore_barrier()
        base = (c * NS + s) * EPT

        @pl.loop(0, WINS)
        def _(w):
            o = base + w * K
            pltpu.sync_copy(src_hbm.at[pl.ds(o, K)], src_v)
            pltpu.sync_copy(et_hbm.at[pl.ds(o, K)], et_v)
            pltpu.sync_copy(dst_hbm.at[pl.ds(o, K)], dst_v)
            pltpu.sync_copy(scl_hbm.at[pl.ds(o, K)], scl_v)
            for j in range(K // L):
                s16 = src_v[pl.ds(j * L, L)]
                t16 = et_v[pl.ds(j * L, L)]
                gidx_v[pl.ds(j * L, L)] = t16 * N + s16
            pltpu.sync_copy(hr_hbm.at[gidx_v], rows_v)

            @pl.loop(0, K)
            def _(i):
                f16 = plsc.load_gather(scl_v, [jnp.full((L,), i, jnp.int32)])
                for j in range(D // L):
                    rows_v[i, pl.ds(j * L, L)] = rows_v[i, pl.ds(j * L, L)] * f16

            pltpu.sync_copy(rows_v, acc_sh.at[dst_v], add=True)

        plsc.subcore_barrier()

        @pl.loop(0, rpt // zrows)
        def _(i):
            pltpu.sync_copy(
                acc_sh.at[pl.ds(s * rpt + i * zrows, zrows)],
                out_hbm.at[c].at[pl.ds(s * rpt + i * zrows, zrows)],
            )

    return k(src, et, dst, scl, hr)[:, :N, :]


def _tc_hr(x, comp, basis):
    def body(comp_ref, x_ref, basis_ref, o_ref):
        r = pl.program_id(0)
        w = comp_ref[r, 0] * basis_ref[0]
        for b in range(1, B):
            w = w + comp_ref[r, b] * basis_ref[b]
        o_ref[...] = jnp.dot(x_ref[...], w, preferred_element_type=jnp.float32)

    return pl.pallas_call(
        body,
        grid=(R,),
        in_specs=[
            pl.BlockSpec(memory_space=pltpu.SMEM),
            pl.BlockSpec((N, D), lambda r: (0, 0)),
            pl.BlockSpec((B, D, D), lambda r: (0, 0, 0)),
        ],
        out_specs=pl.BlockSpec((N, D), lambda r: (r, 0)),
        out_shape=jax.ShapeDtypeStruct((R * N, D), jnp.float32),
    )(comp, x, basis)


def _tc_combine(parts, x, root, bias, g, b, relu):
    def body(p_ref, x_ref, root_ref, bias_ref, g_ref, b_ref, o_ref):
        h = (
            p_ref[0]
            + p_ref[1]
            + jnp.dot(x_ref[...], root_ref[...], preferred_element_type=jnp.float32)
            + bias_ref[...]
        )
        mu = jnp.mean(h, axis=-1, keepdims=True)
        d = h - mu
        var = jnp.mean(d * d, axis=-1, keepdims=True)
        hn = d * lax.rsqrt(var + 1e-5) * g_ref[...] + b_ref[...]
        if relu:
            hn = jnp.maximum(hn, 0.0)
        o_ref[...] = hn

    return pl.pallas_call(
        body,
        out_shape=jax.ShapeDtypeStruct((N, D), jnp.float32),
    )(parts, x, root, bias.reshape(1, D), g.reshape(1, D), b.reshape(1, D))


def kernel(edge_index, edge_type, emb, basis1, comp1, root1, bias1, ln1_g,
           ln1_b, basis2, comp2, root2, bias2, ln2_g, ln2_b):
    src = edge_index[0]
    dst = edge_index[1]
    et = edge_type

    cparts = _sc_counts(dst, et)
    inv = _tc_inv(cparts[:, :N, ::L]).reshape(N * R)
    scl = _sc_scale(dst, et, inv)

    h = emb
    for basis, comp, root, bias, g, bln, relu in (
        (basis1, comp1, root1, bias1, ln1_g, ln1_b, True),
        (basis2, comp2, root2, bias2, ln2_g, ln2_b, False),
    ):
        hr = _tc_hr(h, comp, basis)
        parts = _sc_edge(src, et, dst, scl, hr)
        h = _tc_combine(parts, h, root, bias, g, bln, relu)
    return h


# speedup vs baseline: 14.7416x; 1.8043x over previous
import dataclasses
import functools

import jax
import jax.numpy as jnp
from jax import lax
from jax.experimental import pallas as pl
from jax.experimental.pallas import tpu as pltpu
from jax.experimental.pallas import tpu_sc as plsc

N = 10000
E = 320000
D = 128
R = 8
B = 4

NC = 2
NS = 16
NW = NC * NS
L = 16

K = 80
EPT = E // NW
WINS = EPT // K
CW = 16
SEG = R * N

_mesh = functools.partial(
    plsc.VectorSubcoreMesh, core_axis_name="c", subcore_axis_name="s"
)

_SC_PARAMS = pltpu.CompilerParams()
if "needs_layout_passes" in pltpu.CompilerParams.__dataclass_fields__:
    _SC_PARAMS = dataclasses.replace(_SC_PARAMS, needs_layout_passes=False)


def _sc_counts(dst, et):
    NP = 10240
    zrows = 128

    @functools.partial(
        pl.kernel,
        mesh=_mesh(),
        out_type=jax.ShapeDtypeStruct((NC, NP, D), jnp.float32),
        compiler_params=_SC_PARAMS,
        scratch_types=[
            pltpu.VMEM((K,), jnp.int32),
            pltpu.VMEM((K,), jnp.int32),
            pltpu.VMEM((K, D), jnp.float32),
            pltpu.VMEM((zrows, D), jnp.float32),
            pltpu.VMEM_SHARED((NP, D), jnp.float32),
        ],
    )
    def k(dst_hbm, et_hbm, out_hbm, dst_v, et_v, rows_v, z_v, acc_sh):
        c = lax.axis_index("c")
        s = lax.axis_index("s")

        @pl.loop(0, K)
        def _(i):
            for j in range(D // L):
                rows_v[i, pl.ds(j * L, L)] = jnp.zeros((L,), jnp.float32)

        @pl.loop(0, zrows)
        def _(i):
            for j in range(D // L):
                z_v[i, pl.ds(j * L, L)] = jnp.zeros((L,), jnp.float32)

        rpt = NP // NS

        @pl.loop(0, rpt // zrows)
        def _(i):
            pltpu.sync_copy(z_v, acc_sh.at[pl.ds(s * rpt + i * zrows, zrows)])

        plsc.subcore_barrier()
        base = (c * NS + s) * EPT
        ones16 = jnp.ones((L,), jnp.float32)
        zeros16 = jnp.zeros((L,), jnp.float32)
        iota16 = lax.broadcasted_iota(jnp.int32, (L,), 0)

        @pl.loop(0, WINS)
        def _(w):
            o = base + w * K
            pltpu.sync_copy(dst_hbm.at[pl.ds(o, K)], dst_v)
            pltpu.sync_copy(et_hbm.at[pl.ds(o, K)], et_v)
            for j in range(K // L):
                rid = iota16 + (j * L)
                col = et_v[pl.ds(j * L, L)] * L
                plsc.store_scatter(rows_v, [rid, col], ones16)
            pltpu.sync_copy(rows_v, acc_sh.at[dst_v], add=True)
            for j in range(K // L):
                rid = iota16 + (j * L)
                col = et_v[pl.ds(j * L, L)] * L
                plsc.store_scatter(rows_v, [rid, col], zeros16)

        plsc.subcore_barrier()

        @pl.loop(0, rpt // zrows)
        def _(i):
            pltpu.sync_copy(
                acc_sh.at[pl.ds(s * rpt + i * zrows, zrows)],
                out_hbm.at[c].at[pl.ds(s * rpt + i * zrows, zrows)],
            )

    return k(dst, et)


def _tc_inv(cgrid):
    def body(c_ref, o_ref):
        cnt = c_ref[0] + c_ref[1]
        o_ref[...] = 1.0 / jnp.maximum(cnt, 1.0)

    return pl.pallas_call(
        body,
        out_shape=jax.ShapeDtypeStruct((N, R), jnp.float32),
    )(cgrid)


def _sc_scale(dst, et, inv):
    @functools.partial(
        pl.kernel,
        mesh=_mesh(),
        out_type=jax.ShapeDtypeStruct((E,), jnp.float32),
        compiler_params=_SC_PARAMS,
        scratch_types=[
            pltpu.VMEM((SEG,), jnp.float32),
            pltpu.VMEM((K,), jnp.int32),
            pltpu.VMEM((K,), jnp.int32),
            pltpu.VMEM((K,), jnp.float32),
        ],
    )
    def k(dst_hbm, et_hbm, inv_hbm, out_hbm, inv_v, dst_v, et_v, scl_v):
        c = lax.axis_index("c")
        s = lax.axis_index("s")
        pltpu.sync_copy(inv_hbm, inv_v)
        base = (c * NS + s) * EPT

        @pl.loop(0, WINS)
        def _(w):
            o = base + w * K
            pltpu.sync_copy(dst_hbm.at[pl.ds(o, K)], dst_v)
            pltpu.sync_copy(et_hbm.at[pl.ds(o, K)], et_v)
            for j in range(K // L):
                d16 = dst_v[pl.ds(j * L, L)]
                t16 = et_v[pl.ds(j * L, L)]
                scl_v[pl.ds(j * L, L)] = plsc.load_gather(inv_v, [d16 * R + t16])
            pltpu.sync_copy(scl_v, out_hbm.at[pl.ds(o, K)])

    return k(dst, et, inv)


def _sc_edge(src, et, dst, scl, hr):
    npad = 10240
    wchunk = 128

    @functools.partial(
        pl.kernel,
        mesh=_mesh(),
        out_type=jax.ShapeDtypeStruct((NC, npad, D), jnp.float32),
        compiler_params=_SC_PARAMS,
        scratch_types=[
            pltpu.VMEM((K,), jnp.int32),
            pltpu.VMEM((K,), jnp.int32),
            pltpu.VMEM((K,), jnp.int32),
            pltpu.VMEM((K,), jnp.int32),
            pltpu.VMEM((K,), jnp.float32),
            pltpu.VMEM((K,), jnp.float32),
            pltpu.VMEM((K,), jnp.int32),
            pltpu.VMEM((K,), jnp.int32),
            pltpu.VMEM((K,), jnp.int32),
            pltpu.VMEM((K,), jnp.int32),
            pltpu.VMEM((K, D), jnp.float32),
            pltpu.VMEM((K, D), jnp.float32),
            pltpu.VMEM_SHARED((npad, D), jnp.float32),
            pltpu.SemaphoreType.DMA,
            pltpu.SemaphoreType.DMA,
            pltpu.SemaphoreType.DMA,
            pltpu.SemaphoreType.DMA,
            pltpu.SemaphoreType.DMA,
            pltpu.SemaphoreType.DMA,
        ],
    )
    def k(src_hbm, et_hbm, dst_hbm, scl_hbm, hr_hbm, out_hbm,
          srcb0, srcb1, etb0, etb1, sclb0, sclb1, gidx0, gidx1, dstb0, dstb1,
          rows0, rows1, acc_sh, gsem0, gsem1, ssem0, ssem1, isem0, isem1):
        c = lax.axis_index("c")
        s = lax.axis_index("s")
        wid = c * NS + s
        rpt = npad // NS

        @pl.loop(0, K)
        def _(i):
            for j in range(D // L):
                rows0[i, pl.ds(j * L, L)] = jnp.zeros((L,), jnp.float32)

        @pl.loop(0, rpt // K)
        def _(i):
            pltpu.sync_copy(rows0, acc_sh.at[pl.ds(s * rpt + i * K, K)])

        plsc.subcore_barrier()

        def issue3(w, srcb, etb, sclb, sem):
            o = wid * EPT + w * K
            pltpu.async_copy(src_hbm.at[pl.ds(o, K)], srcb, sem)
            pltpu.async_copy(et_hbm.at[pl.ds(o, K)], etb, sem)
            pltpu.async_copy(scl_hbm.at[pl.ds(o, K)], sclb, sem)

        def wait3(srcb, etb, sclb, sem):
            pltpu.make_async_copy(src_hbm.at[pl.ds(0, K)], srcb, sem).wait()
            pltpu.make_async_copy(et_hbm.at[pl.ds(0, K)], etb, sem).wait()
            pltpu.make_async_copy(scl_hbm.at[pl.ds(0, K)], sclb, sem).wait()

        def issue_dst(w, dstb, sem):
            pltpu.async_copy(dst_hbm.at[pl.ds(wid * EPT + w * K, K)], dstb, sem)

        def wait_dst(dstb, sem):
            pltpu.make_async_copy(dst_hbm.at[pl.ds(0, K)], dstb, sem).wait()

        def compute_gidx(gidx, srcb, etb):
            for j in range(K // L):
                s16 = srcb[pl.ds(j * L, L)]
                t16 = etb[pl.ds(j * L, L)]
                gidx[pl.ds(j * L, L)] = t16 * N + s16

        def issue_gather(gidx, rows, sem):
            pltpu.async_copy(hr_hbm.at[gidx], rows, sem)

        def wait_gather(rows, sem):
            pltpu.make_async_copy(hr_hbm.at[gidx0], rows, sem).wait()

        def issue_scatter(rows, dstb, sem):
            pltpu.async_copy(rows, acc_sh.at[dstb], sem, add=True)

        def wait_scatter(rows, sem):
            pltpu.make_async_copy(rows, acc_sh.at[dstb0], sem).wait()

        def scale_rows(rows, sclb):
            @pl.loop(0, K, unroll=8)
            def _(i):
                f16 = plsc.load_gather(sclb, [jnp.full((L,), i, jnp.int32)])
                for j in range(D // L):
                    rows[i, pl.ds(j * L, L)] = rows[i, pl.ds(j * L, L)] * f16

        issue3(0, srcb0, etb0, sclb0, isem0)
        issue_dst(0, dstb0, isem0)
        wait3(srcb0, etb0, sclb0, isem0)
        compute_gidx(gidx0, srcb0, etb0)
        issue_gather(gidx0, rows0, gsem0)
        issue3(1, srcb1, etb1, sclb1, isem1)
        issue_dst(1, dstb1, isem1)
        wait3(srcb1, etb1, sclb1, isem1)
        compute_gidx(gidx1, srcb1, etb1)
        issue_gather(gidx1, rows1, gsem1)

        @pl.loop(0, (WINS - 1) // 2)
        def _(p):
            w0 = 2 * p
            wait_gather(rows0, gsem0)
            scale_rows(rows0, sclb0)
            wait_dst(dstb0, isem0)
            issue_scatter(rows0, dstb0, ssem0)
            issue3(w0 + 2, srcb0, etb0, sclb0, isem0)
            wait_gather(rows1, gsem1)
            scale_rows(rows1, sclb1)
            wait_dst(dstb1, isem1)
            issue_scatter(rows1, dstb1, ssem1)

            @pl.when(w0 + 3 < WINS)
            def _():
                issue3(w0 + 3, srcb1, etb1, sclb1, isem1)

            wait3(srcb0, etb0, sclb0, isem0)
            compute_gidx(gidx0, srcb0, etb0)
            wait_scatter(rows0, ssem0)
            issue_dst(w0 + 2, dstb0, isem0)
            issue_gather(gidx0, rows0, gsem0)

            @pl.when(w0 + 3 < WINS)
            def _():
                wait3(srcb1, etb1, sclb1, isem1)
                compute_gidx(gidx1, srcb1, etb1)
                wait_scatter(rows1, ssem1)
                issue_dst(w0 + 3, dstb1, isem1)
                issue_gather(gidx1, rows1, gsem1)

        wait_gather(rows0, gsem0)
        wait_scatter(rows1, ssem1)
        scale_rows(rows0, sclb0)
        wait_dst(dstb0, isem0)
        pltpu.sync_copy(rows0, acc_sh.at[dstb0], add=True)

        plsc.subcore_barrier()

        @pl.loop(0, rpt // wchunk)
        def _(i):
            pltpu.sync_copy(
                acc_sh.at[pl.ds(s * rpt + i * wchunk, wchunk)],
                out_hbm.at[c].at[pl.ds(s * rpt + i * wchunk, wchunk)],
            )

    return k(src, et, dst, scl, hr)[:, :N, :]


def _tc_hr(x, comp, basis):
    def body(comp_ref, x_ref, basis_ref, o_ref):
        r = pl.program_id(0)
        w = comp_ref[r, 0] * basis_ref[0]
        for b in range(1, B):
            w = w + comp_ref[r, b] * basis_ref[b]
        o_ref[...] = jnp.dot(x_ref[...], w, preferred_element_type=jnp.float32)

    return pl.pallas_call(
        body,
        grid=(R,),
        in_specs=[
            pl.BlockSpec(memory_space=pltpu.SMEM),
            pl.BlockSpec((N, D), lambda r: (0, 0)),
            pl.BlockSpec((B, D, D), lambda r: (0, 0, 0)),
        ],
        out_specs=pl.BlockSpec((N, D), lambda r: (r, 0)),
        out_shape=jax.ShapeDtypeStruct((R * N, D), jnp.float32),
    )(comp, x, basis)


def _tc_combine(parts, x, root, bias, g, b, relu):
    def body(p_ref, x_ref, root_ref, bias_ref, g_ref, b_ref, o_ref):
        h = (
            p_ref[0]
            + p_ref[1]
            + jnp.dot(x_ref[...], root_ref[...], preferred_element_type=jnp.float32)
            + bias_ref[...]
        )
        mu = jnp.mean(h, axis=-1, keepdims=True)
        d = h - mu
        var = jnp.mean(d * d, axis=-1, keepdims=True)
        hn = d * lax.rsqrt(var + 1e-5) * g_ref[...] + b_ref[...]
        if relu:
            hn = jnp.maximum(hn, 0.0)
        o_ref[...] = hn

    return pl.pallas_call(
        body,
        out_shape=jax.ShapeDtypeStruct((N, D), jnp.float32),
    )(parts, x, root, bias.reshape(1, D), g.reshape(1, D), b.reshape(1, D))


def kernel(edge_index, edge_type, emb, basis1, comp1, root1, bias1, ln1_g,
           ln1_b, basis2, comp2, root2, bias2, ln2_g, ln2_b):
    src = edge_index[0]
    dst = edge_index[1]
    et = edge_type

    cparts = _sc_counts(dst, et)
    inv = _tc_inv(cparts[:, :N, ::L]).reshape(N * R)
    scl = _sc_scale(dst, et, inv)


    h = emb
    for basis, comp, root, bias, g, bln, relu in (
        (basis1, comp1, root1, bias1, ln1_g, ln1_b, True),
        (basis2, comp2, root2, bias2, ln2_g, ln2_b, False),
    ):
        hr = _tc_hr(h, comp, basis)
        parts = _sc_edge(src, et, dst, scl, hr)
        h = _tc_combine(parts, h, root, bias, g, bln, relu)
    return h


# speedup vs baseline: 19.3719x; 1.3141x over previous
import dataclasses
import functools

import jax
import jax.numpy as jnp
from jax import lax
from jax.experimental import pallas as pl
from jax.experimental.pallas import tpu as pltpu
from jax.experimental.pallas import tpu_sc as plsc

N = 10000
E = 320000
D = 128
R = 8
B = 4

NC = 2
NS = 16
NW = NC * NS
L = 16

K = 80
EPT = E // NW
WINS = EPT // K
CW = 16
SEG = R * N

_mesh = functools.partial(
    plsc.VectorSubcoreMesh, core_axis_name="c", subcore_axis_name="s"
)

_SC_PARAMS = pltpu.CompilerParams()
if "needs_layout_passes" in pltpu.CompilerParams.__dataclass_fields__:
    _SC_PARAMS = dataclasses.replace(_SC_PARAMS, needs_layout_passes=False)


def _sc_counts(dst, et):
    NP = 10240
    zrows = 128

    @functools.partial(
        pl.kernel,
        mesh=_mesh(),
        out_type=jax.ShapeDtypeStruct((NC, NP, D), jnp.float32),
        compiler_params=_SC_PARAMS,
        scratch_types=[
            pltpu.VMEM((K,), jnp.int32),
            pltpu.VMEM((K,), jnp.int32),
            pltpu.VMEM((K,), jnp.int32),
            pltpu.VMEM((K,), jnp.int32),
            pltpu.VMEM((K,), jnp.int32),
            pltpu.VMEM((K,), jnp.int32),
            pltpu.VMEM((K, D), jnp.float32),
            pltpu.VMEM((K, D), jnp.float32),
            pltpu.VMEM_SHARED((NP, D), jnp.float32),
            pltpu.SemaphoreType.DMA,
            pltpu.SemaphoreType.DMA,
            pltpu.SemaphoreType.DMA,
            pltpu.SemaphoreType.DMA,
            pltpu.SemaphoreType.DMA,
            pltpu.SemaphoreType.DMA,
        ],
    )
    def k(dst_hbm, et_hbm, out_hbm, dstb0, dstb1, etb0, etb1, colb0, colb1,
          rows0, rows1, acc_sh, esem0, esem1, dsem0, dsem1, ssem0, ssem1):
        c = lax.axis_index("c")
        s = lax.axis_index("s")
        wid = c * NS + s
        rpt = NP // NS
        iota16 = lax.broadcasted_iota(jnp.int32, (L,), 0)
        ones16 = jnp.ones((L,), jnp.float32)
        zeros16 = jnp.zeros((L,), jnp.float32)

        for rows in (rows0, rows1):
            @pl.loop(0, K)
            def _(i):
                for j in range(D // L):
                    rows[i, pl.ds(j * L, L)] = jnp.zeros((L,), jnp.float32)

        @pl.loop(0, rpt // K)
        def _(i):
            pltpu.sync_copy(rows0, acc_sh.at[pl.ds(s * rpt + i * K, K)])

        plsc.subcore_barrier()

        def issue_et(w, etb, sem):
            pltpu.async_copy(et_hbm.at[pl.ds(wid * EPT + w * K, K)], etb, sem)

        def issue_dst(w, dstb, sem):
            pltpu.async_copy(dst_hbm.at[pl.ds(wid * EPT + w * K, K)], dstb, sem)

        def wait_load(buf, sem):
            pltpu.make_async_copy(et_hbm.at[pl.ds(0, K)], buf, sem).wait()

        def wait_scatter(rows, sem):
            pltpu.make_async_copy(rows, acc_sh.at[dstb0], sem).wait()

        def clear_rows(rows, colb):
            for j in range(K // L):
                rid = iota16 + (j * L)
                plsc.store_scatter(rows, [rid, colb[pl.ds(j * L, L)]], zeros16)

        def build_rows(rows, etb, colb):
            for j in range(K // L):
                rid = iota16 + (j * L)
                col = etb[pl.ds(j * L, L)] * L
                plsc.store_scatter(rows, [rid, col], ones16)
                colb[pl.ds(j * L, L)] = col

        def half(p, w, etb, dstb, colb, rows, esem, dsem, ssem):
            @pl.when(p > 0)
            def _():
                wait_scatter(rows, ssem)

            issue_dst(w, dstb, dsem)

            @pl.when(p > 0)
            def _():
                clear_rows(rows, colb)

            wait_load(etb, esem)
            build_rows(rows, etb, colb)

            @pl.when(w + 2 < WINS)
            def _():
                issue_et(w + 2, etb, esem)

            wait_load(dstb, dsem)
            pltpu.async_copy(rows, acc_sh.at[dstb], ssem, add=True)

        issue_et(0, etb0, esem0)
        issue_et(1, etb1, esem1)

        @pl.loop(0, (WINS - 1) // 2)
        def _(p):
            half(p, 2 * p, etb0, dstb0, colb0, rows0, esem0, dsem0, ssem0)
            half(p, 2 * p + 1, etb1, dstb1, colb1, rows1, esem1, dsem1, ssem1)

        wait_scatter(rows0, ssem0)
        issue_dst(WINS - 1, dstb0, dsem0)
        clear_rows(rows0, colb0)
        wait_load(etb0, esem0)
        build_rows(rows0, etb0, colb0)
        wait_load(dstb0, dsem0)
        pltpu.sync_copy(rows0, acc_sh.at[dstb0], add=True)
        wait_scatter(rows1, ssem1)

        plsc.subcore_barrier()

        @pl.loop(0, rpt // zrows)
        def _(i):
            pltpu.sync_copy(
                acc_sh.at[pl.ds(s * rpt + i * zrows, zrows)],
                out_hbm.at[c].at[pl.ds(s * rpt + i * zrows, zrows)],
            )

    return k(dst, et)


def _tc_inv(cgrid):
    def body(c_ref, o_ref):
        cnt = c_ref[0] + c_ref[1]
        o_ref[...] = 1.0 / jnp.maximum(cnt, 1.0)

    return pl.pallas_call(
        body,
        out_shape=jax.ShapeDtypeStruct((N, R), jnp.float32),
    )(cgrid)


def _sc_scale(dst, et, inv):
    @functools.partial(
        pl.kernel,
        mesh=_mesh(),
        out_type=jax.ShapeDtypeStruct((E,), jnp.float32),
        compiler_params=_SC_PARAMS,
        scratch_types=[
            pltpu.VMEM((SEG,), jnp.float32),
            pltpu.VMEM((K,), jnp.int32),
            pltpu.VMEM((K,), jnp.int32),
            pltpu.VMEM((K,), jnp.int32),
            pltpu.VMEM((K,), jnp.int32),
            pltpu.VMEM((K,), jnp.float32),
            pltpu.VMEM((K,), jnp.float32),
            pltpu.SemaphoreType.DMA,
            pltpu.SemaphoreType.DMA,
            pltpu.SemaphoreType.DMA,
            pltpu.SemaphoreType.DMA,
        ],
    )
    def k(dst_hbm, et_hbm, inv_hbm, out_hbm, inv_v,
          dstb0, dstb1, etb0, etb1, sclb0, sclb1, isem0, isem1, osem0, osem1):
        c = lax.axis_index("c")
        s = lax.axis_index("s")
        wid = c * NS + s
        pltpu.sync_copy(inv_hbm, inv_v)

        def issue2(w, dstb, etb, sem):
            o = wid * EPT + w * K
            pltpu.async_copy(dst_hbm.at[pl.ds(o, K)], dstb, sem)
            pltpu.async_copy(et_hbm.at[pl.ds(o, K)], etb, sem)

        def wait2(dstb, etb, sem):
            pltpu.make_async_copy(dst_hbm.at[pl.ds(0, K)], dstb, sem).wait()
            pltpu.make_async_copy(et_hbm.at[pl.ds(0, K)], etb, sem).wait()

        def wait_store(sclb, sem):
            pltpu.make_async_copy(sclb, out_hbm.at[pl.ds(0, K)], sem).wait()

        def half(p, w, dstb, etb, sclb, isem, osem):
            wait2(dstb, etb, isem)

            @pl.when(p > 0)
            def _():
                wait_store(sclb, osem)

            for j in range(K // L):
                d16 = dstb[pl.ds(j * L, L)]
                t16 = etb[pl.ds(j * L, L)]
                sclb[pl.ds(j * L, L)] = plsc.load_gather(inv_v, [d16 * R + t16])
            pltpu.async_copy(sclb, out_hbm.at[pl.ds(wid * EPT + w * K, K)], osem)

            @pl.when(w + 2 < WINS)
            def _():
                issue2(w + 2, dstb, etb, isem)

        issue2(0, dstb0, etb0, isem0)
        issue2(1, dstb1, etb1, isem1)

        @pl.loop(0, (WINS - 1) // 2)
        def _(p):
            half(p, 2 * p, dstb0, etb0, sclb0, isem0, osem0)
            half(p, 2 * p + 1, dstb1, etb1, sclb1, isem1, osem1)

        wait2(dstb0, etb0, isem0)
        wait_store(sclb0, osem0)
        for j in range(K // L):
            d16 = dstb0[pl.ds(j * L, L)]
            t16 = etb0[pl.ds(j * L, L)]
            sclb0[pl.ds(j * L, L)] = plsc.load_gather(inv_v, [d16 * R + t16])
        pltpu.sync_copy(sclb0, out_hbm.at[pl.ds(wid * EPT + (WINS - 1) * K, K)])
        wait_store(sclb1, osem1)

    return k(dst, et, inv)


def _sc_edge(src, et, dst, scl, hr):
    npad = 10240
    wchunk = 128

    @functools.partial(
        pl.kernel,
        mesh=_mesh(),
        out_type=jax.ShapeDtypeStruct((NC, npad, D), jnp.float32),
        compiler_params=_SC_PARAMS,
        scratch_types=[
            pltpu.VMEM((K,), jnp.int32),
            pltpu.VMEM((K,), jnp.int32),
            pltpu.VMEM((K,), jnp.int32),
            pltpu.VMEM((K,), jnp.int32),
            pltpu.VMEM((K,), jnp.float32),
            pltpu.VMEM((K,), jnp.float32),
            pltpu.VMEM((K,), jnp.int32),
            pltpu.VMEM((K,), jnp.int32),
            pltpu.VMEM((K,), jnp.int32),
            pltpu.VMEM((K,), jnp.int32),
            pltpu.VMEM((K, D), jnp.float32),
            pltpu.VMEM((K, D), jnp.float32),
            pltpu.VMEM_SHARED((npad, D), jnp.float32),
            pltpu.SemaphoreType.DMA,
            pltpu.SemaphoreType.DMA,
            pltpu.SemaphoreType.DMA,
            pltpu.SemaphoreType.DMA,
            pltpu.SemaphoreType.DMA,
            pltpu.SemaphoreType.DMA,
        ],
    )
    def k(src_hbm, et_hbm, dst_hbm, scl_hbm, hr_hbm, out_hbm,
          srcb0, srcb1, etb0, etb1, sclb0, sclb1, gidx0, gidx1, dstb0, dstb1,
          rows0, rows1, acc_sh, gsem0, gsem1, ssem0, ssem1, isem0, isem1):
        c = lax.axis_index("c")
        s = lax.axis_index("s")
        wid = c * NS + s
        rpt = npad // NS

        @pl.loop(0, K)
        def _(i):
            for j in range(D // L):
                rows0[i, pl.ds(j * L, L)] = jnp.zeros((L,), jnp.float32)

        @pl.loop(0, rpt // K)
        def _(i):
            pltpu.sync_copy(rows0, acc_sh.at[pl.ds(s * rpt + i * K, K)])

        plsc.subcore_barrier()

        def issue3(w, srcb, etb, sclb, sem):
            o = wid * EPT + w * K
            pltpu.async_copy(src_hbm.at[pl.ds(o, K)], srcb, sem)
            pltpu.async_copy(et_hbm.at[pl.ds(o, K)], etb, sem)
            pltpu.async_copy(scl_hbm.at[pl.ds(o, K)], sclb, sem)

        def wait3(srcb, etb, sclb, sem):
            pltpu.make_async_copy(src_hbm.at[pl.ds(0, K)], srcb, sem).wait()
            pltpu.make_async_copy(et_hbm.at[pl.ds(0, K)], etb, sem).wait()
            pltpu.make_async_copy(scl_hbm.at[pl.ds(0, K)], sclb, sem).wait()

        def issue_dst(w, dstb, sem):
            pltpu.async_copy(dst_hbm.at[pl.ds(wid * EPT + w * K, K)], dstb, sem)

        def wait_dst(dstb, sem):
            pltpu.make_async_copy(dst_hbm.at[pl.ds(0, K)], dstb, sem).wait()

        def compute_gidx(gidx, srcb, etb):
            for j in range(K // L):
                s16 = srcb[pl.ds(j * L, L)]
                t16 = etb[pl.ds(j * L, L)]
                gidx[pl.ds(j * L, L)] = t16 * N + s16

        def issue_gather(gidx, rows, sem):
            pltpu.async_copy(hr_hbm.at[gidx], rows, sem)

        def wait_gather(rows, sem):
            pltpu.make_async_copy(hr_hbm.at[gidx0], rows, sem).wait()

        def issue_scatter(rows, dstb, sem):
            pltpu.async_copy(rows, acc_sh.at[dstb], sem, add=True)

        def wait_scatter(rows, sem):
            pltpu.make_async_copy(rows, acc_sh.at[dstb0], sem).wait()

        def scale_rows(rows, sclb):
            @pl.loop(0, K, unroll=8)
            def _(i):
                f16 = plsc.load_gather(sclb, [jnp.full((L,), i, jnp.int32)])
                for j in range(D // L):
                    rows[i, pl.ds(j * L, L)] = rows[i, pl.ds(j * L, L)] * f16

        issue3(0, srcb0, etb0, sclb0, isem0)
        issue_dst(0, dstb0, isem0)
        wait3(srcb0, etb0, sclb0, isem0)
        compute_gidx(gidx0, srcb0, etb0)
        issue_gather(gidx0, rows0, gsem0)
        issue3(1, srcb1, etb1, sclb1, isem1)
        issue_dst(1, dstb1, isem1)
        wait3(srcb1, etb1, sclb1, isem1)
        compute_gidx(gidx1, srcb1, etb1)
        issue_gather(gidx1, rows1, gsem1)

        @pl.loop(0, (WINS - 1) // 2)
        def _(p):
            w0 = 2 * p
            wait_gather(rows0, gsem0)
            scale_rows(rows0, sclb0)
            wait_dst(dstb0, isem0)
            issue_scatter(rows0, dstb0, ssem0)
            issue3(w0 + 2, srcb0, etb0, sclb0, isem0)
            wait_gather(rows1, gsem1)
            scale_rows(rows1, sclb1)
            wait_dst(dstb1, isem1)
            issue_scatter(rows1, dstb1, ssem1)

            @pl.when(w0 + 3 < WINS)
            def _():
                issue3(w0 + 3, srcb1, etb1, sclb1, isem1)

            wait3(srcb0, etb0, sclb0, isem0)
            compute_gidx(gidx0, srcb0, etb0)
            wait_scatter(rows0, ssem0)
            issue_dst(w0 + 2, dstb0, isem0)
            issue_gather(gidx0, rows0, gsem0)

            @pl.when(w0 + 3 < WINS)
            def _():
                wait3(srcb1, etb1, sclb1, isem1)
                compute_gidx(gidx1, srcb1, etb1)
                wait_scatter(rows1, ssem1)
                issue_dst(w0 + 3, dstb1, isem1)
                issue_gather(gidx1, rows1, gsem1)

        wait_gather(rows0, gsem0)
        wait_scatter(rows1, ssem1)
        scale_rows(rows0, sclb0)
        wait_dst(dstb0, isem0)
        pltpu.sync_copy(rows0, acc_sh.at[dstb0], add=True)

        plsc.subcore_barrier()

        @pl.loop(0, rpt // wchunk)
        def _(i):
            pltpu.sync_copy(
                acc_sh.at[pl.ds(s * rpt + i * wchunk, wchunk)],
                out_hbm.at[c].at[pl.ds(s * rpt + i * wchunk, wchunk)],
            )

    return k(src, et, dst, scl, hr)[:, :N, :]


def _tc_hr(x, comp, basis):
    def body(comp_ref, x_ref, basis_ref, o_ref):
        r = pl.program_id(0)
        w = comp_ref[r, 0] * basis_ref[0]
        for b in range(1, B):
            w = w + comp_ref[r, b] * basis_ref[b]
        o_ref[...] = jnp.dot(x_ref[...], w, preferred_element_type=jnp.float32)

    return pl.pallas_call(
        body,
        grid=(R,),
        in_specs=[
            pl.BlockSpec(memory_space=pltpu.SMEM),
            pl.BlockSpec((N, D), lambda r: (0, 0)),
            pl.BlockSpec((B, D, D), lambda r: (0, 0, 0)),
        ],
        out_specs=pl.BlockSpec((N, D), lambda r: (r, 0)),
        out_shape=jax.ShapeDtypeStruct((R * N, D), jnp.float32),
    )(comp, x, basis)


def _tc_combine(parts, x, root, bias, g, b, relu):
    def body(p_ref, x_ref, root_ref, bias_ref, g_ref, b_ref, o_ref):
        h = (
            p_ref[0]
            + p_ref[1]
            + jnp.dot(x_ref[...], root_ref[...], preferred_element_type=jnp.float32)
            + bias_ref[...]
        )
        mu = jnp.mean(h, axis=-1, keepdims=True)
        d = h - mu
        var = jnp.mean(d * d, axis=-1, keepdims=True)
        hn = d * lax.rsqrt(var + 1e-5) * g_ref[...] + b_ref[...]
        if relu:
            hn = jnp.maximum(hn, 0.0)
        o_ref[...] = hn

    return pl.pallas_call(
        body,
        out_shape=jax.ShapeDtypeStruct((N, D), jnp.float32),
    )(parts, x, root, bias.reshape(1, D), g.reshape(1, D), b.reshape(1, D))


def kernel(edge_index, edge_type, emb, basis1, comp1, root1, bias1, ln1_g,
           ln1_b, basis2, comp2, root2, bias2, ln2_g, ln2_b):
    src = edge_index[0]
    dst = edge_index[1]
    et = edge_type

    cparts = _sc_counts(dst, et)
    inv = _tc_inv(cparts[:, :N, ::L]).reshape(N * R)
    scl = _sc_scale(dst, et, inv)


    h = emb
    for basis, comp, root, bias, g, bln, relu in (
        (basis1, comp1, root1, bias1, ln1_g, ln1_b, True),
        (basis2, comp2, root2, bias2, ln2_g, ln2_b, False),
    ):
        hr = _tc_hr(h, comp, basis)
        parts = _sc_edge(src, et, dst, scl, hr)
        h = _tc_combine(parts, h, root, bias, g, bln, relu)
    return h


# speedup vs baseline: 19.7490x; 1.0195x over previous
import dataclasses
import functools

import jax
import jax.numpy as jnp
from jax import lax
from jax.experimental import pallas as pl
from jax.experimental.pallas import tpu as pltpu
from jax.experimental.pallas import tpu_sc as plsc

N = 10000
E = 320000
D = 128
R = 8
B = 4

NC = 2
NS = 16
NW = NC * NS
L = 16

K = 80
EPT = E // NW
WINS = EPT // K
CW = 16
SEG = R * N

_mesh = functools.partial(
    plsc.VectorSubcoreMesh, core_axis_name="c", subcore_axis_name="s"
)

_SC_PARAMS = pltpu.CompilerParams()
if "needs_layout_passes" in pltpu.CompilerParams.__dataclass_fields__:
    _SC_PARAMS = dataclasses.replace(_SC_PARAMS, needs_layout_passes=False)


def _sc_counts(dst, et):
    NP = 10240
    zrows = 128

    @functools.partial(
        pl.kernel,
        mesh=_mesh(),
        out_type=jax.ShapeDtypeStruct((NC, NP, D), jnp.float32),
        compiler_params=_SC_PARAMS,
        scratch_types=[
            pltpu.VMEM((K,), jnp.int32),
            pltpu.VMEM((K,), jnp.int32),
            pltpu.VMEM((K,), jnp.int32),
            pltpu.VMEM((K,), jnp.int32),
            pltpu.VMEM((K,), jnp.int32),
            pltpu.VMEM((K,), jnp.int32),
            pltpu.VMEM((K, D), jnp.float32),
            pltpu.VMEM((K, D), jnp.float32),
            pltpu.VMEM_SHARED((NP, D), jnp.float32),
            pltpu.SemaphoreType.DMA,
            pltpu.SemaphoreType.DMA,
            pltpu.SemaphoreType.DMA,
            pltpu.SemaphoreType.DMA,
            pltpu.SemaphoreType.DMA,
            pltpu.SemaphoreType.DMA,
        ],
    )
    def k(dst_hbm, et_hbm, out_hbm, dstb0, dstb1, etb0, etb1, colb0, colb1,
          rows0, rows1, acc_sh, esem0, esem1, dsem0, dsem1, ssem0, ssem1):
        c = lax.axis_index("c")
        s = lax.axis_index("s")
        wid = c * NS + s
        rpt = NP // NS
        iota16 = lax.broadcasted_iota(jnp.int32, (L,), 0)
        ones16 = jnp.ones((L,), jnp.float32)
        zeros16 = jnp.zeros((L,), jnp.float32)

        for rows in (rows0, rows1):
            @pl.loop(0, K)
            def _(i):
                for j in range(D // L):
                    rows[i, pl.ds(j * L, L)] = jnp.zeros((L,), jnp.float32)

        @pl.loop(0, rpt // K)
        def _(i):
            pltpu.sync_copy(rows0, acc_sh.at[pl.ds(s * rpt + i * K, K)])

        plsc.subcore_barrier()

        def issue_et(w, etb, sem):
            pltpu.async_copy(et_hbm.at[pl.ds(wid * EPT + w * K, K)], etb, sem)

        def issue_dst(w, dstb, sem):
            pltpu.async_copy(dst_hbm.at[pl.ds(wid * EPT + w * K, K)], dstb, sem)

        def wait_load(buf, sem):
            pltpu.make_async_copy(et_hbm.at[pl.ds(0, K)], buf, sem).wait()

        def wait_scatter(rows, sem):
            pltpu.make_async_copy(rows, acc_sh.at[dstb0], sem).wait()

        def clear_rows(rows, colb):
            for j in range(K // L):
                rid = iota16 + (j * L)
                plsc.store_scatter(rows, [rid, colb[pl.ds(j * L, L)]], zeros16)

        def build_rows(rows, etb, colb):
            for j in range(K // L):
                rid = iota16 + (j * L)
                col = etb[pl.ds(j * L, L)] * L
                plsc.store_scatter(rows, [rid, col], ones16)
                colb[pl.ds(j * L, L)] = col

        def half(p, w, etb, dstb, colb, rows, esem, dsem, ssem):
            @pl.when(p > 0)
            def _():
                wait_scatter(rows, ssem)

            issue_dst(w, dstb, dsem)

            @pl.when(p > 0)
            def _():
                clear_rows(rows, colb)

            wait_load(etb, esem)
            build_rows(rows, etb, colb)

            @pl.when(w + 2 < WINS)
            def _():
                issue_et(w + 2, etb, esem)

            wait_load(dstb, dsem)
            pltpu.async_copy(rows, acc_sh.at[dstb], ssem, add=True)

        issue_et(0, etb0, esem0)
        issue_et(1, etb1, esem1)

        @pl.loop(0, (WINS - 1) // 2)
        def _(p):
            half(p, 2 * p, etb0, dstb0, colb0, rows0, esem0, dsem0, ssem0)
            half(p, 2 * p + 1, etb1, dstb1, colb1, rows1, esem1, dsem1, ssem1)

        wait_scatter(rows0, ssem0)
        issue_dst(WINS - 1, dstb0, dsem0)
        clear_rows(rows0, colb0)
        wait_load(etb0, esem0)
        build_rows(rows0, etb0, colb0)
        wait_load(dstb0, dsem0)
        pltpu.sync_copy(rows0, acc_sh.at[dstb0], add=True)
        wait_scatter(rows1, ssem1)

        plsc.subcore_barrier()

        @pl.loop(0, rpt // zrows)
        def _(i):
            pltpu.sync_copy(
                acc_sh.at[pl.ds(s * rpt + i * zrows, zrows)],
                out_hbm.at[c].at[pl.ds(s * rpt + i * zrows, zrows)],
            )

    return k(dst, et)


def _tc_inv(cgrid):
    def body(c_ref, o_ref):
        cnt = c_ref[0] + c_ref[1]
        o_ref[...] = 1.0 / jnp.maximum(cnt, 1.0)

    return pl.pallas_call(
        body,
        out_shape=jax.ShapeDtypeStruct((N, R), jnp.float32),
    )(cgrid)


def _sc_scale(dst, et, inv):
    @functools.partial(
        pl.kernel,
        mesh=_mesh(),
        out_type=jax.ShapeDtypeStruct((E,), jnp.float32),
        compiler_params=_SC_PARAMS,
        scratch_types=[
            pltpu.VMEM((SEG,), jnp.float32),
            pltpu.VMEM((K,), jnp.int32),
            pltpu.VMEM((K,), jnp.int32),
            pltpu.VMEM((K,), jnp.int32),
            pltpu.VMEM((K,), jnp.int32),
            pltpu.VMEM((K,), jnp.float32),
            pltpu.VMEM((K,), jnp.float32),
            pltpu.SemaphoreType.DMA,
            pltpu.SemaphoreType.DMA,
            pltpu.SemaphoreType.DMA,
            pltpu.SemaphoreType.DMA,
        ],
    )
    def k(dst_hbm, et_hbm, inv_hbm, out_hbm, inv_v,
          dstb0, dstb1, etb0, etb1, sclb0, sclb1, isem0, isem1, osem0, osem1):
        c = lax.axis_index("c")
        s = lax.axis_index("s")
        wid = c * NS + s
        pltpu.sync_copy(inv_hbm, inv_v)

        def issue2(w, dstb, etb, sem):
            o = wid * EPT + w * K
            pltpu.async_copy(dst_hbm.at[pl.ds(o, K)], dstb, sem)
            pltpu.async_copy(et_hbm.at[pl.ds(o, K)], etb, sem)

        def wait2(dstb, etb, sem):
            pltpu.make_async_copy(dst_hbm.at[pl.ds(0, K)], dstb, sem).wait()
            pltpu.make_async_copy(et_hbm.at[pl.ds(0, K)], etb, sem).wait()

        def wait_store(sclb, sem):
            pltpu.make_async_copy(sclb, out_hbm.at[pl.ds(0, K)], sem).wait()

        def half(p, w, dstb, etb, sclb, isem, osem):
            wait2(dstb, etb, isem)

            @pl.when(p > 0)
            def _():
                wait_store(sclb, osem)

            for j in range(K // L):
                d16 = dstb[pl.ds(j * L, L)]
                t16 = etb[pl.ds(j * L, L)]
                sclb[pl.ds(j * L, L)] = plsc.load_gather(inv_v, [d16 * R + t16])
            pltpu.async_copy(sclb, out_hbm.at[pl.ds(wid * EPT + w * K, K)], osem)

            @pl.when(w + 2 < WINS)
            def _():
                issue2(w + 2, dstb, etb, isem)

        issue2(0, dstb0, etb0, isem0)
        issue2(1, dstb1, etb1, isem1)

        @pl.loop(0, (WINS - 1) // 2)
        def _(p):
            half(p, 2 * p, dstb0, etb0, sclb0, isem0, osem0)
            half(p, 2 * p + 1, dstb1, etb1, sclb1, isem1, osem1)

        wait2(dstb0, etb0, isem0)
        wait_store(sclb0, osem0)
        for j in range(K // L):
            d16 = dstb0[pl.ds(j * L, L)]
            t16 = etb0[pl.ds(j * L, L)]
            sclb0[pl.ds(j * L, L)] = plsc.load_gather(inv_v, [d16 * R + t16])
        pltpu.sync_copy(sclb0, out_hbm.at[pl.ds(wid * EPT + (WINS - 1) * K, K)])
        wait_store(sclb1, osem1)

    return k(dst, et, inv)


def _sc_edge(src, et, dst, scl, hr):
    npad = 10240
    wchunk = 128

    @functools.partial(
        pl.kernel,
        mesh=_mesh(),
        out_type=jax.ShapeDtypeStruct((NC, npad, D), jnp.float32),
        compiler_params=_SC_PARAMS,
        scratch_types=(
            [pltpu.VMEM((K,), jnp.int32)] * 4
            + [pltpu.VMEM((K,), jnp.int32)] * 4
            + [pltpu.VMEM((K,), jnp.float32)] * 4
            + [pltpu.VMEM((K,), jnp.int32)] * 4
            + [pltpu.VMEM((K,), jnp.int32)] * 4
            + [pltpu.VMEM((K, D), jnp.float32)] * 4
            + [pltpu.VMEM_SHARED((npad, D), jnp.float32)]
            + [pltpu.SemaphoreType.DMA] * 16
        ),
    )
    def k(src_hbm, et_hbm, dst_hbm, scl_hbm, hr_hbm, out_hbm, *scr):
        srcb = scr[0:4]
        etb = scr[4:8]
        sclb = scr[8:12]
        gidxb = scr[12:16]
        dstb = scr[16:20]
        rowsb = scr[20:24]
        acc_sh = scr[24]
        gsem = scr[25:29]
        ssem = scr[29:33]
        isem = scr[33:37]
        dsem = scr[37:41]
        c = lax.axis_index("c")
        s = lax.axis_index("s")
        wid = c * NS + s
        rpt = npad // NS

        @pl.loop(0, K)
        def _(i):
            for j in range(D // L):
                rowsb[0][i, pl.ds(j * L, L)] = jnp.zeros((L,), jnp.float32)

        @pl.loop(0, rpt // K)
        def _(i):
            pltpu.sync_copy(rowsb[0], acc_sh.at[pl.ds(s * rpt + i * K, K)])

        plsc.subcore_barrier()

        def issue3(w, srcb, etb, sclb, sem):
            o = wid * EPT + w * K
            pltpu.async_copy(src_hbm.at[pl.ds(o, K)], srcb, sem)
            pltpu.async_copy(et_hbm.at[pl.ds(o, K)], etb, sem)
            pltpu.async_copy(scl_hbm.at[pl.ds(o, K)], sclb, sem)

        def wait3(srcb, etb, sclb, sem):
            pltpu.make_async_copy(src_hbm.at[pl.ds(0, K)], srcb, sem).wait()
            pltpu.make_async_copy(et_hbm.at[pl.ds(0, K)], etb, sem).wait()
            pltpu.make_async_copy(scl_hbm.at[pl.ds(0, K)], sclb, sem).wait()

        def issue_dst(w, dstb, sem):
            pltpu.async_copy(dst_hbm.at[pl.ds(wid * EPT + w * K, K)], dstb, sem)

        def wait_dst(dstb, sem):
            pltpu.make_async_copy(dst_hbm.at[pl.ds(0, K)], dstb, sem).wait()

        def compute_gidx(gidx, srcb, etb):
            for j in range(K // L):
                s16 = srcb[pl.ds(j * L, L)]
                t16 = etb[pl.ds(j * L, L)]
                gidx[pl.ds(j * L, L)] = t16 * N + s16

        def issue_gather(gidx, rows, sem):
            pltpu.async_copy(hr_hbm.at[gidx], rows, sem)

        def wait_gather(rows, sem):
            pltpu.make_async_copy(hr_hbm.at[gidxb[0]], rows, sem).wait()

        def issue_scatter(rows, dst_b, sem):
            pltpu.async_copy(rows, acc_sh.at[dst_b], sem, add=True)

        def wait_scatter(rows, sem):
            pltpu.make_async_copy(rows, acc_sh.at[dstb[0]], sem).wait()

        def scale_rows(rows, scl_b):
            @pl.loop(0, K, unroll=8)
            def _(i):
                f16 = plsc.load_gather(scl_b, [jnp.full((L,), i, jnp.int32)])
                for j in range(D // L):
                    rows[i, pl.ds(j * L, L)] = rows[i, pl.ds(j * L, L)] * f16

        for b in range(4):
            issue3(b, srcb[b], etb[b], sclb[b], isem[b])
            wait3(srcb[b], etb[b], sclb[b], isem[b])
            compute_gidx(gidxb[b], srcb[b], etb[b])
            issue_dst(b, dstb[b], dsem[b])
            issue_gather(gidxb[b], rowsb[b], gsem[b])

        @pl.loop(0, (WINS - 1) // 4)
        def _(q):
            w = 4 * q
            for b in range(4):
                wait_gather(rowsb[b], gsem[b])
                scale_rows(rowsb[b], sclb[b])
                wait_dst(dstb[b], dsem[b])
                issue_scatter(rowsb[b], dstb[b], ssem[b])

                @pl.when(w + b + 4 < WINS)
                def _():
                    issue3(w + b + 4, srcb[b], etb[b], sclb[b], isem[b])

            for b in range(4):
                @pl.when(w + b + 4 < WINS)
                def _():
                    wait3(srcb[b], etb[b], sclb[b], isem[b])
                    compute_gidx(gidxb[b], srcb[b], etb[b])

                wait_scatter(rowsb[b], ssem[b])

                @pl.when(w + b + 4 < WINS)
                def _():
                    issue_dst(w + b + 4, dstb[b], dsem[b])
                    issue_gather(gidxb[b], rowsb[b], gsem[b])

        wait_gather(rowsb[0], gsem[0])
        scale_rows(rowsb[0], sclb[0])
        wait_dst(dstb[0], dsem[0])
        pltpu.sync_copy(rowsb[0], acc_sh.at[dstb[0]], add=True)

        plsc.subcore_barrier()

        @pl.loop(0, rpt // wchunk)
        def _(i):
            pltpu.sync_copy(
                acc_sh.at[pl.ds(s * rpt + i * wchunk, wchunk)],
                out_hbm.at[c].at[pl.ds(s * rpt + i * wchunk, wchunk)],
            )

    return k(src, et, dst, scl, hr)[:, :N, :]


def _tc_hr(x, comp, basis):
    def body(comp_ref, x_ref, basis_ref, o_ref):
        r = pl.program_id(0)
        w = comp_ref[r, 0] * basis_ref[0]
        for b in range(1, B):
            w = w + comp_ref[r, b] * basis_ref[b]
        o_ref[...] = jnp.dot(x_ref[...], w, preferred_element_type=jnp.float32)

    return pl.pallas_call(
        body,
        grid=(R,),
        in_specs=[
            pl.BlockSpec(memory_space=pltpu.SMEM),
            pl.BlockSpec((N, D), lambda r: (0, 0)),
            pl.BlockSpec((B, D, D), lambda r: (0, 0, 0)),
        ],
        out_specs=pl.BlockSpec((N, D), lambda r: (r, 0)),
        out_shape=jax.ShapeDtypeStruct((R * N, D), jnp.float32),
    )(comp, x, basis)


def _tc_combine(parts, x, root, bias, g, b, relu):
    def body(p_ref, x_ref, root_ref, bias_ref, g_ref, b_ref, o_ref):
        h = (
            p_ref[0]
            + p_ref[1]
            + jnp.dot(x_ref[...], root_ref[...], preferred_element_type=jnp.float32)
            + bias_ref[...]
        )
        mu = jnp.mean(h, axis=-1, keepdims=True)
        d = h - mu
        var = jnp.mean(d * d, axis=-1, keepdims=True)
        hn = d * lax.rsqrt(var + 1e-5) * g_ref[...] + b_ref[...]
        if relu:
            hn = jnp.maximum(hn, 0.0)
        o_ref[...] = hn

    return pl.pallas_call(
        body,
        out_shape=jax.ShapeDtypeStruct((N, D), jnp.float32),
    )(parts, x, root, bias.reshape(1, D), g.reshape(1, D), b.reshape(1, D))


def kernel(edge_index, edge_type, emb, basis1, comp1, root1, bias1, ln1_g,
           ln1_b, basis2, comp2, root2, bias2, ln2_g, ln2_b):
    src = edge_index[0]
    dst = edge_index[1]
    et = edge_type

    cparts = _sc_counts(dst, et)
    inv = _tc_inv(cparts[:, :N, ::L]).reshape(N * R)
    scl = _sc_scale(dst, et, inv)


    h = emb
    for basis, comp, root, bias, g, bln, relu in (
        (basis1, comp1, root1, bias1, ln1_g, ln1_b, True),
        (basis2, comp2, root2, bias2, ln2_g, ln2_b, False),
    ):
        hr = _tc_hr(h, comp, basis)
        parts = _sc_edge(src, et, dst, scl, hr)
        h = _tc_combine(parts, h, root, bias, g, bln, relu)
    return h


# speedup vs baseline: 24.5396x; 1.2426x over previous
import dataclasses
import functools

import jax
import jax.numpy as jnp
from jax import lax
from jax.experimental import pallas as pl
from jax.experimental.pallas import tpu as pltpu
from jax.experimental.pallas import tpu_sc as plsc

N = 10000
E = 320000
D = 128
R = 8
B = 4

NC = 2
NS = 16
NW = NC * NS
L = 16

K = 80
EPT = E // NW
WINS = EPT // K
CW = 16
SEG = R * N

_mesh = functools.partial(
    plsc.VectorSubcoreMesh, core_axis_name="c", subcore_axis_name="s"
)

_SC_PARAMS = pltpu.CompilerParams()
if "needs_layout_passes" in pltpu.CompilerParams.__dataclass_fields__:
    _SC_PARAMS = dataclasses.replace(_SC_PARAMS, needs_layout_passes=False)


def _sc_counts(dst, et):
    NP = 10240
    zrows = 128

    @functools.partial(
        pl.kernel,
        mesh=_mesh(),
        out_type=jax.ShapeDtypeStruct((NC, NP, D), jnp.float32),
        compiler_params=_SC_PARAMS,
        scratch_types=[
            pltpu.VMEM((K,), jnp.int32),
            pltpu.VMEM((K,), jnp.int32),
            pltpu.VMEM((K,), jnp.int32),
            pltpu.VMEM((K,), jnp.int32),
            pltpu.VMEM((K,), jnp.int32),
            pltpu.VMEM((K,), jnp.int32),
            pltpu.VMEM((K, D), jnp.float32),
            pltpu.VMEM((K, D), jnp.float32),
            pltpu.VMEM_SHARED((NP, D), jnp.float32),
            pltpu.SemaphoreType.DMA,
            pltpu.SemaphoreType.DMA,
            pltpu.SemaphoreType.DMA,
            pltpu.SemaphoreType.DMA,
            pltpu.SemaphoreType.DMA,
            pltpu.SemaphoreType.DMA,
        ],
    )
    def k(dst_hbm, et_hbm, out_hbm, dstb0, dstb1, etb0, etb1, colb0, colb1,
          rows0, rows1, acc_sh, esem0, esem1, dsem0, dsem1, ssem0, ssem1):
        c = lax.axis_index("c")
        s = lax.axis_index("s")
        wid = c * NS + s
        rpt = NP // NS
        iota16 = lax.broadcasted_iota(jnp.int32, (L,), 0)
        ones16 = jnp.ones((L,), jnp.float32)
        zeros16 = jnp.zeros((L,), jnp.float32)

        for rows in (rows0, rows1):
            @pl.loop(0, K)
            def _(i):
                for j in range(D // L):
                    rows[i, pl.ds(j * L, L)] = jnp.zeros((L,), jnp.float32)

        @pl.loop(0, rpt // K)
        def _(i):
            pltpu.sync_copy(rows0, acc_sh.at[pl.ds(s * rpt + i * K, K)])

        plsc.subcore_barrier()

        def issue_et(w, etb, sem):
            pltpu.async_copy(et_hbm.at[pl.ds(wid * EPT + w * K, K)], etb, sem)

        def issue_dst(w, dstb, sem):
            pltpu.async_copy(dst_hbm.at[pl.ds(wid * EPT + w * K, K)], dstb, sem)

        def wait_load(buf, sem):
            pltpu.make_async_copy(et_hbm.at[pl.ds(0, K)], buf, sem).wait()

        def wait_scatter(rows, sem):
            pltpu.make_async_copy(rows, acc_sh.at[dstb0], sem).wait()

        def clear_rows(rows, colb):
            for j in range(K // L):
                rid = iota16 + (j * L)
                plsc.store_scatter(rows, [rid, colb[pl.ds(j * L, L)]], zeros16)

        def build_rows(rows, etb, colb):
            for j in range(K // L):
                rid = iota16 + (j * L)
                col = etb[pl.ds(j * L, L)] * L
                plsc.store_scatter(rows, [rid, col], ones16)
                colb[pl.ds(j * L, L)] = col

        def half(p, w, etb, dstb, colb, rows, esem, dsem, ssem):
            @pl.when(p > 0)
            def _():
                wait_scatter(rows, ssem)

            issue_dst(w, dstb, dsem)

            @pl.when(p > 0)
            def _():
                clear_rows(rows, colb)

            wait_load(etb, esem)
            build_rows(rows, etb, colb)

            @pl.when(w + 2 < WINS)
            def _():
                issue_et(w + 2, etb, esem)

            wait_load(dstb, dsem)
            pltpu.async_copy(rows, acc_sh.at[dstb], ssem, add=True)

        issue_et(0, etb0, esem0)
        issue_et(1, etb1, esem1)

        @pl.loop(0, (WINS - 1) // 2)
        def _(p):
            half(p, 2 * p, etb0, dstb0, colb0, rows0, esem0, dsem0, ssem0)
            half(p, 2 * p + 1, etb1, dstb1, colb1, rows1, esem1, dsem1, ssem1)

        wait_scatter(rows0, ssem0)
        issue_dst(WINS - 1, dstb0, dsem0)
        clear_rows(rows0, colb0)
        wait_load(etb0, esem0)
        build_rows(rows0, etb0, colb0)
        wait_load(dstb0, dsem0)
        pltpu.sync_copy(rows0, acc_sh.at[dstb0], add=True)
        wait_scatter(rows1, ssem1)

        plsc.subcore_barrier()

        @pl.loop(0, rpt // zrows)
        def _(i):
            pltpu.sync_copy(
                acc_sh.at[pl.ds(s * rpt + i * zrows, zrows)],
                out_hbm.at[c].at[pl.ds(s * rpt + i * zrows, zrows)],
            )

    return k(dst, et)


def _tc_inv(cgrid):
    def body(c_ref, o_ref):
        cnt = c_ref[0] + c_ref[1]
        o_ref[...] = 1.0 / jnp.maximum(cnt, 1.0)

    return pl.pallas_call(
        body,
        out_shape=jax.ShapeDtypeStruct((N, R), jnp.float32),
    )(cgrid)


def _sc_scale(dst, et, inv):
    @functools.partial(
        pl.kernel,
        mesh=_mesh(),
        out_type=jax.ShapeDtypeStruct((E,), jnp.float32),
        compiler_params=_SC_PARAMS,
        scratch_types=[
            pltpu.VMEM((SEG,), jnp.float32),
            pltpu.VMEM((K,), jnp.int32),
            pltpu.VMEM((K,), jnp.int32),
            pltpu.VMEM((K,), jnp.int32),
            pltpu.VMEM((K,), jnp.int32),
            pltpu.VMEM((K,), jnp.float32),
            pltpu.VMEM((K,), jnp.float32),
            pltpu.SemaphoreType.DMA,
            pltpu.SemaphoreType.DMA,
            pltpu.SemaphoreType.DMA,
            pltpu.SemaphoreType.DMA,
        ],
    )
    def k(dst_hbm, et_hbm, inv_hbm, out_hbm, inv_v,
          dstb0, dstb1, etb0, etb1, sclb0, sclb1, isem0, isem1, osem0, osem1):
        c = lax.axis_index("c")
        s = lax.axis_index("s")
        wid = c * NS + s
        pltpu.sync_copy(inv_hbm, inv_v)

        def issue2(w, dstb, etb, sem):
            o = wid * EPT + w * K
            pltpu.async_copy(dst_hbm.at[pl.ds(o, K)], dstb, sem)
            pltpu.async_copy(et_hbm.at[pl.ds(o, K)], etb, sem)

        def wait2(dstb, etb, sem):
            pltpu.make_async_copy(dst_hbm.at[pl.ds(0, K)], dstb, sem).wait()
            pltpu.make_async_copy(et_hbm.at[pl.ds(0, K)], etb, sem).wait()

        def wait_store(sclb, sem):
            pltpu.make_async_copy(sclb, out_hbm.at[pl.ds(0, K)], sem).wait()

        def half(p, w, dstb, etb, sclb, isem, osem):
            wait2(dstb, etb, isem)

            @pl.when(p > 0)
            def _():
                wait_store(sclb, osem)

            for j in range(K // L):
                d16 = dstb[pl.ds(j * L, L)]
                t16 = etb[pl.ds(j * L, L)]
                sclb[pl.ds(j * L, L)] = plsc.load_gather(inv_v, [d16 * R + t16])
            pltpu.async_copy(sclb, out_hbm.at[pl.ds(wid * EPT + w * K, K)], osem)

            @pl.when(w + 2 < WINS)
            def _():
                issue2(w + 2, dstb, etb, isem)

        issue2(0, dstb0, etb0, isem0)
        issue2(1, dstb1, etb1, isem1)

        @pl.loop(0, (WINS - 1) // 2)
        def _(p):
            half(p, 2 * p, dstb0, etb0, sclb0, isem0, osem0)
            half(p, 2 * p + 1, dstb1, etb1, sclb1, isem1, osem1)

        wait2(dstb0, etb0, isem0)
        wait_store(sclb0, osem0)
        for j in range(K // L):
            d16 = dstb0[pl.ds(j * L, L)]
            t16 = etb0[pl.ds(j * L, L)]
            sclb0[pl.ds(j * L, L)] = plsc.load_gather(inv_v, [d16 * R + t16])
        pltpu.sync_copy(sclb0, out_hbm.at[pl.ds(wid * EPT + (WINS - 1) * K, K)])
        wait_store(sclb1, osem1)

    return k(dst, et, inv)


def _sc_edge(src, et, dst, scl, hr):
    npad = 10240
    wchunk = 128

    @functools.partial(
        pl.kernel,
        mesh=_mesh(),
        out_type=jax.ShapeDtypeStruct((NC, npad, D), jnp.float32),
        compiler_params=_SC_PARAMS,
        scratch_types=(
            [pltpu.VMEM((K,), jnp.int32)] * 4
            + [pltpu.VMEM((K,), jnp.int32)] * 4
            + [pltpu.VMEM((K,), jnp.float32)] * 4
            + [pltpu.VMEM((K,), jnp.int32)] * 4
            + [pltpu.VMEM((K,), jnp.int32)] * 4
            + [pltpu.VMEM((K, D), jnp.float32)] * 4
            + [pltpu.VMEM_SHARED((npad, D), jnp.float32)]
            + [pltpu.SemaphoreType.DMA] * 16
        ),
    )
    def k(src_hbm, et_hbm, dst_hbm, scl_hbm, hr_hbm, out_hbm, *scr):
        srcb = scr[0:4]
        etb = scr[4:8]
        sclb = scr[8:12]
        gidxb = scr[12:16]
        dstb = scr[16:20]
        rowsb = scr[20:24]
        acc_sh = scr[24]
        gsem = scr[25:29]
        ssem = scr[29:33]
        isem = scr[33:37]
        dsem = scr[37:41]
        c = lax.axis_index("c")
        s = lax.axis_index("s")
        wid = c * NS + s
        rpt = npad // NS

        @pl.loop(0, K)
        def _(i):
            for j in range(D // L):
                rowsb[0][i, pl.ds(j * L, L)] = jnp.zeros((L,), jnp.float32)

        @pl.loop(0, rpt // K)
        def _(i):
            pltpu.sync_copy(rowsb[0], acc_sh.at[pl.ds(s * rpt + i * K, K)])

        plsc.subcore_barrier()

        def issue3(w, srcb, etb, sclb, sem):
            o = wid * EPT + w * K
            pltpu.async_copy(src_hbm.at[pl.ds(o, K)], srcb, sem)
            pltpu.async_copy(et_hbm.at[pl.ds(o, K)], etb, sem)
            pltpu.async_copy(scl_hbm.at[pl.ds(o, K)], sclb, sem)

        def wait3(srcb, etb, sclb, sem):
            pltpu.make_async_copy(src_hbm.at[pl.ds(0, K)], srcb, sem).wait()
            pltpu.make_async_copy(et_hbm.at[pl.ds(0, K)], etb, sem).wait()
            pltpu.make_async_copy(scl_hbm.at[pl.ds(0, K)], sclb, sem).wait()

        def issue_dst(w, dstb, sem):
            pltpu.async_copy(dst_hbm.at[pl.ds(wid * EPT + w * K, K)], dstb, sem)

        def wait_dst(dstb, sem):
            pltpu.make_async_copy(dst_hbm.at[pl.ds(0, K)], dstb, sem).wait()

        def compute_gidx(gidx, srcb, etb):
            for j in range(K // L):
                s16 = srcb[pl.ds(j * L, L)]
                t16 = etb[pl.ds(j * L, L)]
                gidx[pl.ds(j * L, L)] = t16 * N + s16

        def issue_gather(gidx, rows, sem):
            pltpu.async_copy(hr_hbm.at[gidx], rows, sem)

        def wait_gather(rows, sem):
            pltpu.make_async_copy(hr_hbm.at[gidxb[0]], rows, sem).wait()

        def issue_scatter(rows, dst_b, sem):
            pltpu.async_copy(rows, acc_sh.at[dst_b], sem, add=True)

        def wait_scatter(rows, sem):
            pltpu.make_async_copy(rows, acc_sh.at[dstb[0]], sem).wait()

        def scale_rows(rows, scl_b):
            @functools.partial(plsc.parallel_loop, 0, K, unroll=8)
            def _(i):
                f16 = plsc.load_gather(scl_b, [jnp.full((L,), i, jnp.int32)])
                for j in range(D // L):
                    rows[i, pl.ds(j * L, L)] = rows[i, pl.ds(j * L, L)] * f16

        for b in range(4):
            issue3(b, srcb[b], etb[b], sclb[b], isem[b])
            wait3(srcb[b], etb[b], sclb[b], isem[b])
            compute_gidx(gidxb[b], srcb[b], etb[b])
            issue_dst(b, dstb[b], dsem[b])
            issue_gather(gidxb[b], rowsb[b], gsem[b])

        @pl.loop(0, (WINS - 1) // 4)
        def _(q):
            w = 4 * q
            for b in range(4):
                wait_gather(rowsb[b], gsem[b])
                scale_rows(rowsb[b], sclb[b])
                wait_dst(dstb[b], dsem[b])
                issue_scatter(rowsb[b], dstb[b], ssem[b])

                @pl.when(w + b + 4 < WINS)
                def _():
                    issue3(w + b + 4, srcb[b], etb[b], sclb[b], isem[b])

            for b in range(4):
                @pl.when(w + b + 4 < WINS)
                def _():
                    wait3(srcb[b], etb[b], sclb[b], isem[b])
                    compute_gidx(gidxb[b], srcb[b], etb[b])

                wait_scatter(rowsb[b], ssem[b])

                @pl.when(w + b + 4 < WINS)
                def _():
                    issue_dst(w + b + 4, dstb[b], dsem[b])
                    issue_gather(gidxb[b], rowsb[b], gsem[b])

        wait_gather(rowsb[0], gsem[0])
        scale_rows(rowsb[0], sclb[0])
        wait_dst(dstb[0], dsem[0])
        pltpu.sync_copy(rowsb[0], acc_sh.at[dstb[0]], add=True)

        plsc.subcore_barrier()

        @pl.loop(0, rpt // wchunk)
        def _(i):
            pltpu.sync_copy(
                acc_sh.at[pl.ds(s * rpt + i * wchunk, wchunk)],
                out_hbm.at[c].at[pl.ds(s * rpt + i * wchunk, wchunk)],
            )

    return k(src, et, dst, scl, hr)[:, :N, :]


def _tc_hr(x, comp, basis):
    def body(comp_ref, x_ref, basis_ref, o_ref):
        r = pl.program_id(0)
        w = comp_ref[r, 0] * basis_ref[0]
        for b in range(1, B):
            w = w + comp_ref[r, b] * basis_ref[b]
        o_ref[...] = jnp.dot(x_ref[...], w, preferred_element_type=jnp.float32)

    return pl.pallas_call(
        body,
        grid=(R,),
        in_specs=[
            pl.BlockSpec(memory_space=pltpu.SMEM),
            pl.BlockSpec((N, D), lambda r: (0, 0)),
            pl.BlockSpec((B, D, D), lambda r: (0, 0, 0)),
        ],
        out_specs=pl.BlockSpec((N, D), lambda r: (r, 0)),
        out_shape=jax.ShapeDtypeStruct((R * N, D), jnp.float32),
    )(comp, x, basis)


def _tc_combine(parts, x, root, bias, g, b, relu):
    def body(p_ref, x_ref, root_ref, bias_ref, g_ref, b_ref, o_ref):
        h = (
            p_ref[0]
            + p_ref[1]
            + jnp.dot(x_ref[...], root_ref[...], preferred_element_type=jnp.float32)
            + bias_ref[...]
        )
        mu = jnp.mean(h, axis=-1, keepdims=True)
        d = h - mu
        var = jnp.mean(d * d, axis=-1, keepdims=True)
        hn = d * lax.rsqrt(var + 1e-5) * g_ref[...] + b_ref[...]
        if relu:
            hn = jnp.maximum(hn, 0.0)
        o_ref[...] = hn

    return pl.pallas_call(
        body,
        out_shape=jax.ShapeDtypeStruct((N, D), jnp.float32),
    )(parts, x, root, bias.reshape(1, D), g.reshape(1, D), b.reshape(1, D))


def kernel(edge_index, edge_type, emb, basis1, comp1, root1, bias1, ln1_g,
           ln1_b, basis2, comp2, root2, bias2, ln2_g, ln2_b):
    src = edge_index[0]
    dst = edge_index[1]
    et = edge_type

    cparts = _sc_counts(dst, et)
    inv = _tc_inv(cparts[:, :N, ::L]).reshape(N * R)
    scl = _sc_scale(dst, et, inv)


    h = emb
    for basis, comp, root, bias, g, bln, relu in (
        (basis1, comp1, root1, bias1, ln1_g, ln1_b, True),
        (basis2, comp2, root2, bias2, ln2_g, ln2_b, False),
    ):
        hr = _tc_hr(h, comp, basis)
        parts = _sc_edge(src, et, dst, scl, hr)
        h = _tc_combine(parts, h, root, bias, g, bln, relu)
    return h
